# baseline (speedup 1.0000x reference)
import functools

import jax
import jax.numpy as jnp
from jax import lax
from jax.experimental import pallas as pl
from jax.experimental.pallas import tpu as pltpu
from jax.experimental.pallas import tpu_sc as plsc

HIDDEN = 256
N_NODES = 10000
DEG = 16
N_CLASSES = 104

NW = 32
MSGS = N_NODES * DEG
CHUNK = 128
NCHUNK = 40
MPW = NCHUNK * CHUNK
MSGS_PAD = NW * MPW
ROWS_PAD = MSGS_PAD // DEG

@functools.cache
def _get_sc_gather():
    mesh = plsc.VectorSubcoreMesh(core_axis_name="c", subcore_axis_name="s")

    @functools.partial(
        pl.kernel,
        mesh=mesh,
        out_type=jax.ShapeDtypeStruct((MSGS_PAD, HIDDEN // 2), jnp.int32),
        scratch_types=[
            pltpu.VMEM((NCHUNK, CHUNK), jnp.int32),
            pltpu.VMEM((N_NODES,), jnp.int32),
            pltpu.VMEM((CHUNK, HIDDEN // 2), jnp.int32),
            pltpu.VMEM((CHUNK, HIDDEN // 2), jnp.int32),
            pltpu.SemaphoreType.DMA,
            pltpu.SemaphoreType.DMA,
        ],
        compiler_params=pltpu.CompilerParams(needs_layout_passes=False),
    )
    def sc_gather(tok_hbm, nbr_hbm, emb_hbm, out_hbm, idx2, tok_v, buf_a,
                  buf_b, sem_a, sem_b):
        wid = lax.axis_index("s") * 2 + lax.axis_index("c")
        base = wid * MPW
        pltpu.sync_copy(nbr_hbm.at[wid], idx2)
        pltpu.sync_copy(tok_hbm, tok_v)

        def compose_row(r, carry):
            for b in range(CHUNK // 16):
                sl = pl.ds(b * 16, 16)
                nb = idx2[r, sl]
                idx2[r, sl] = plsc.load_gather(tok_v, [nb])
            return carry

        lax.fori_loop(0, NCHUNK, compose_row, 0)

        def gather_pair(k, carry):
            r0 = 2 * k
            r1 = r0 + 1
            cp_a = pltpu.async_copy(emb_hbm.at[idx2.at[r0]], buf_a, sem_a)
            cp_b = pltpu.async_copy(emb_hbm.at[idx2.at[r1]], buf_b, sem_b)
            cp_a.wait()
            pltpu.sync_copy(buf_a, out_hbm.at[pl.ds(base + r0 * CHUNK, CHUNK)])
            cp_b.wait()
            pltpu.sync_copy(buf_b, out_hbm.at[pl.ds(base + r1 * CHUNK, CHUNK)])
            return carry

        lax.fori_loop(0, NCHUNK // 2, gather_pair, 0)

    return sc_gather


ROWS_PER_BLOCK = 400


def _gru_body(msg_ref, wih_ref, whh_ref, bih_ref, bhh_ref, lng_ref, lnb_ref,
              fcw_ref, fcb_ref, out_ref):
    wih = wih_ref[...]
    whh = whh_ref[...]
    bih = bih_ref[...]
    bhh = bhh_ref[...]
    h = msg_ref[:, DEG - 1, :].astype(jnp.float32)
    for t in range(DEG - 1):
        x = msg_ref[:, t, :]
        gi = jnp.dot(x, wih, preferred_element_type=jnp.float32) + bih
        gh = jnp.dot(h, whh, preferred_element_type=jnp.float32) + bhh
        r = jax.nn.sigmoid(gi[:, :HIDDEN] + gh[:, :HIDDEN])
        z = jax.nn.sigmoid(gi[:, HIDDEN:2 * HIDDEN] + gh[:, HIDDEN:2 * HIDDEN])
        n = jnp.tanh(gi[:, 2 * HIDDEN:] + r * gh[:, 2 * HIDDEN:])
        h = (1.0 - z) * n + z * h
    mu = jnp.mean(h, axis=-1, keepdims=True)
    var = jnp.mean((h - mu) * (h - mu), axis=-1, keepdims=True)
    ln = (h - mu) * lax.rsqrt(var + 1e-5) * lng_ref[...] + lnb_ref[...]
    out_ref[...] = jnp.dot(ln, fcw_ref[...],
                           preferred_element_type=jnp.float32) + fcb_ref[...]


_gru_call = pl.pallas_call(
    _gru_body,
    grid=(N_NODES // ROWS_PER_BLOCK,),
    in_specs=[
        pl.BlockSpec((ROWS_PER_BLOCK, DEG, HIDDEN), lambda i: (i, 0, 0)),
        pl.BlockSpec((HIDDEN, 3 * HIDDEN), lambda i: (0, 0)),
        pl.BlockSpec((HIDDEN, 3 * HIDDEN), lambda i: (0, 0)),
        pl.BlockSpec((1, 3 * HIDDEN), lambda i: (0, 0)),
        pl.BlockSpec((1, 3 * HIDDEN), lambda i: (0, 0)),
        pl.BlockSpec((1, HIDDEN), lambda i: (0, 0)),
        pl.BlockSpec((1, HIDDEN), lambda i: (0, 0)),
        pl.BlockSpec((HIDDEN, N_CLASSES), lambda i: (0, 0)),
        pl.BlockSpec((1, N_CLASSES), lambda i: (0, 0)),
    ],
    out_specs=pl.BlockSpec((ROWS_PER_BLOCK, N_CLASSES), lambda i: (i, 0)),
    out_shape=jax.ShapeDtypeStruct((N_NODES, N_CLASSES), jnp.float32),
    compiler_params=pltpu.CompilerParams(
        dimension_semantics=("arbitrary",),
    ),
)


def kernel(token_ids, neighbor_idx, emb, W_ih, W_hh, b_ih, b_hh, ln_g, ln_b,
           fc_W, fc_b):
    nbr_flat = neighbor_idx.reshape(-1).astype(jnp.int32)
    nbr_flat = jnp.concatenate(
        [nbr_flat, jnp.zeros((MSGS_PAD - MSGS,), jnp.int32)])
    nbr3 = nbr_flat.reshape(NW, NCHUNK, CHUNK)
    emb_b = emb.astype(jnp.bfloat16).reshape(emb.shape[0], HIDDEN // 2, 2)
    emb_i = lax.bitcast_convert_type(emb_b, jnp.int32)
    msg_i = _get_sc_gather()(token_ids.astype(jnp.int32), nbr3, emb_i)
    msg_b = lax.bitcast_convert_type(msg_i, jnp.bfloat16)
    msg = msg_b.reshape(ROWS_PAD, DEG, HIDDEN)
    out = _gru_call(
        msg,
        W_ih.T.astype(jnp.bfloat16), W_hh.T,
        b_ih.reshape(1, -1), b_hh.reshape(1, -1),
        ln_g.reshape(1, -1), ln_b.reshape(1, -1),
        fc_W.T, fc_b.reshape(1, -1),
    )
    return out


# speedup vs baseline: 4.9965x; 4.9965x over previous
import functools

import jax
import jax.numpy as jnp
from jax import lax
from jax.experimental import pallas as pl
from jax.experimental.pallas import tpu as pltpu
from jax.experimental.pallas import tpu_sc as plsc

HIDDEN = 256
N_NODES = 10000
DEG = 16
N_CLASSES = 104

NW = 32
MSGS = N_NODES * DEG
CHUNK = 64
NCHUNK = 80
NBUF = 4
MPW = NCHUNK * CHUNK
MSGS_PAD = NW * MPW
ROWS_PAD = MSGS_PAD // DEG

@functools.cache
def _get_sc_gather():
    mesh = plsc.VectorSubcoreMesh(core_axis_name="c", subcore_axis_name="s")

    @functools.partial(
        pl.kernel,
        mesh=mesh,
        out_type=jax.ShapeDtypeStruct((MSGS_PAD, HIDDEN), jnp.float32),
        scratch_types=[
            pltpu.VMEM((NCHUNK, CHUNK), jnp.int32),
            pltpu.VMEM((N_NODES,), jnp.int32),
            pltpu.VMEM((NBUF, CHUNK, HIDDEN), jnp.float32),
            pltpu.SemaphoreType.DMA,
            pltpu.SemaphoreType.DMA,
            pltpu.SemaphoreType.DMA,
            pltpu.SemaphoreType.DMA,
        ],
        compiler_params=pltpu.CompilerParams(needs_layout_passes=False),
    )
    def sc_gather(tok_hbm, nbr_hbm, emb_hbm, out_hbm, idx2, tok_v, ring,
                  sem0, sem1, sem2, sem3):
        sems = (sem0, sem1, sem2, sem3)
        wid = lax.axis_index("s") * 2 + lax.axis_index("c")
        base = wid * MPW
        pltpu.sync_copy(nbr_hbm.at[wid], idx2)
        pltpu.sync_copy(tok_hbm, tok_v)

        def compose_row(r, carry):
            for b in range(CHUNK // 16):
                sl = pl.ds(b * 16, 16)
                nb = idx2[r, sl]
                idx2[r, sl] = plsc.load_gather(tok_v, [nb])
            return carry

        lax.fori_loop(0, NCHUNK, compose_row, 0)

        for b in range(NBUF):
            pltpu.async_copy(emb_hbm.at[idx2.at[b]], ring.at[b], sems[b])

        def ring_step(k, carry):
            for b in range(NBUF):
                r = NBUF * k + b
                pltpu.make_async_copy(
                    emb_hbm.at[pl.ds(0, CHUNK)], ring.at[b], sems[b]).wait()
                pltpu.sync_copy(ring.at[b],
                                out_hbm.at[pl.ds(base + r * CHUNK, CHUNK)])
                nr = r + NBUF

                @pl.when(nr < NCHUNK)
                def _():
                    pltpu.async_copy(emb_hbm.at[idx2.at[nr]], ring.at[b],
                                     sems[b])
            return carry

        lax.fori_loop(0, NCHUNK // NBUF, ring_step, 0)

    return sc_gather


ROWS_PER_BLOCK = 400


def _gru_body(msg_ref, wih_ref, whh_ref, bih_ref, bhh_ref, lng_ref, lnb_ref,
              fcw_ref, fcb_ref, out_ref):
    wih = wih_ref[...]
    whh = whh_ref[...]
    bih = bih_ref[...]
    bhh = bhh_ref[...]
    h = msg_ref[:, DEG - 1, :]
    for t in range(DEG - 1):
        x = msg_ref[:, t, :]
        gi = jnp.dot(x, wih, preferred_element_type=jnp.float32) + bih
        gh = jnp.dot(h, whh, preferred_element_type=jnp.float32) + bhh
        r = jax.nn.sigmoid(gi[:, :HIDDEN] + gh[:, :HIDDEN])
        z = jax.nn.sigmoid(gi[:, HIDDEN:2 * HIDDEN] + gh[:, HIDDEN:2 * HIDDEN])
        n = jnp.tanh(gi[:, 2 * HIDDEN:] + r * gh[:, 2 * HIDDEN:])
        h = (1.0 - z) * n + z * h
    mu = jnp.mean(h, axis=-1, keepdims=True)
    var = jnp.mean((h - mu) * (h - mu), axis=-1, keepdims=True)
    ln = (h - mu) * lax.rsqrt(var + 1e-5) * lng_ref[...] + lnb_ref[...]
    out_ref[...] = jnp.dot(ln, fcw_ref[...],
                           preferred_element_type=jnp.float32) + fcb_ref[...]


_gru_call = pl.pallas_call(
    _gru_body,
    grid=(N_NODES // ROWS_PER_BLOCK,),
    in_specs=[
        pl.BlockSpec((ROWS_PER_BLOCK, DEG, HIDDEN), lambda i: (i, 0, 0)),
        pl.BlockSpec((HIDDEN, 3 * HIDDEN), lambda i: (0, 0)),
        pl.BlockSpec((HIDDEN, 3 * HIDDEN), lambda i: (0, 0)),
        pl.BlockSpec((1, 3 * HIDDEN), lambda i: (0, 0)),
        pl.BlockSpec((1, 3 * HIDDEN), lambda i: (0, 0)),
        pl.BlockSpec((1, HIDDEN), lambda i: (0, 0)),
        pl.BlockSpec((1, HIDDEN), lambda i: (0, 0)),
        pl.BlockSpec((HIDDEN, N_CLASSES), lambda i: (0, 0)),
        pl.BlockSpec((1, N_CLASSES), lambda i: (0, 0)),
    ],
    out_specs=pl.BlockSpec((ROWS_PER_BLOCK, N_CLASSES), lambda i: (i, 0)),
    out_shape=jax.ShapeDtypeStruct((N_NODES, N_CLASSES), jnp.float32),
    compiler_params=pltpu.CompilerParams(
        dimension_semantics=("arbitrary",),
    ),
)


def kernel(token_ids, neighbor_idx, emb, W_ih, W_hh, b_ih, b_hh, ln_g, ln_b,
           fc_W, fc_b):
    nbr_flat = neighbor_idx.reshape(-1).astype(jnp.int32)
    nbr_flat = jnp.concatenate(
        [nbr_flat, jnp.zeros((MSGS_PAD - MSGS,), jnp.int32)])
    nbr3 = nbr_flat.reshape(NW, NCHUNK, CHUNK)
    msg_flat = _get_sc_gather()(token_ids.astype(jnp.int32), nbr3, emb)
    msg = msg_flat.reshape(ROWS_PAD, DEG, HIDDEN)
    out = _gru_call(
        msg,
        W_ih.T, W_hh.T,
        b_ih.reshape(1, -1), b_hh.reshape(1, -1),
        ln_g.reshape(1, -1), ln_b.reshape(1, -1),
        fc_W.T, fc_b.reshape(1, -1),
    )
    return out


# speedup vs baseline: 5.4995x; 1.1007x over previous
import functools

import jax
import jax.numpy as jnp
from jax import lax
from jax.experimental import pallas as pl
from jax.experimental.pallas import tpu as pltpu
from jax.experimental.pallas import tpu_sc as plsc

HIDDEN = 256
N_NODES = 10000
DEG = 16
N_CLASSES = 104

NW = 32
MSGS = N_NODES * DEG
CHUNK = 64
NBUF = 4
CHUNKS_PER_S = 160
CHUNKS_FAST = 120
CHUNKS_SLOW = 40
NCHUNKS_TOT = 16 * CHUNKS_PER_S
MSGS_PAD = NCHUNKS_TOT * CHUNK
IDX_PAD_CHUNKS = NCHUNKS_TOT + CHUNKS_FAST - CHUNKS_SLOW
ROWS_PAD = MSGS_PAD // DEG

@functools.cache
def _get_sc_gather():
    mesh = plsc.VectorSubcoreMesh(core_axis_name="c", subcore_axis_name="s")

    @functools.partial(
        pl.kernel,
        mesh=mesh,
        out_type=jax.ShapeDtypeStruct((MSGS_PAD, HIDDEN), jnp.float32),
        scratch_types=[
            pltpu.VMEM((CHUNKS_FAST, CHUNK), jnp.int32),
            pltpu.VMEM((N_NODES,), jnp.int32),
            pltpu.VMEM((NBUF, CHUNK, HIDDEN), jnp.float32),
            pltpu.SemaphoreType.DMA,
            pltpu.SemaphoreType.DMA,
            pltpu.SemaphoreType.DMA,
            pltpu.SemaphoreType.DMA,
        ],
        compiler_params=pltpu.CompilerParams(needs_layout_passes=False),
    )
    def sc_gather(tok_hbm, nbr_hbm, emb_hbm, out_hbm, idx2, tok_v, ring,
                  sem0, sem1, sem2, sem3):
        sems = (sem0, sem1, sem2, sem3)
        c = lax.axis_index("c")
        s = lax.axis_index("s")
        gstart = s * CHUNKS_PER_S + c * CHUNKS_FAST
        nck = jnp.where(c == 0, CHUNKS_FAST, CHUNKS_SLOW)
        pltpu.sync_copy(nbr_hbm.at[pl.ds(gstart, CHUNKS_FAST)], idx2)
        pltpu.sync_copy(tok_hbm, tok_v)

        def compose_row(r, carry):
            for b in range(CHUNK // 16):
                sl = pl.ds(b * 16, 16)
                nb = idx2[r, sl]
                idx2[r, sl] = plsc.load_gather(tok_v, [nb])
            return carry

        lax.fori_loop(0, nck, compose_row, 0)

        base = gstart * CHUNK
        for b in range(NBUF):
            pltpu.async_copy(emb_hbm.at[idx2.at[b]], ring.at[b], sems[b])

        def ring_step(k, carry):
            for b in range(NBUF):
                r = NBUF * k + b
                pltpu.make_async_copy(
                    emb_hbm.at[pl.ds(0, CHUNK)], ring.at[b], sems[b]).wait()
                pltpu.sync_copy(ring.at[b],
                                out_hbm.at[pl.ds(base + r * CHUNK, CHUNK)])
                nr = r + NBUF

                @pl.when(nr < nck)
                def _():
                    pltpu.async_copy(emb_hbm.at[idx2.at[nr]], ring.at[b],
                                     sems[b])
            return carry

        lax.fori_loop(0, nck // NBUF, ring_step, 0)

    return sc_gather


ROWS_PER_BLOCK = 400


def _gru_body(msg_ref, wih_ref, whh_ref, bih_ref, bhh_ref, lng_ref, lnb_ref,
              fcw_ref, fcb_ref, out_ref):
    wih = wih_ref[...]
    whh = whh_ref[...]
    bih = bih_ref[...]
    bhh = bhh_ref[...]
    h = msg_ref[:, DEG - 1, :]
    for t in range(DEG - 1):
        x = msg_ref[:, t, :]
        gi = jnp.dot(x, wih, preferred_element_type=jnp.float32) + bih
        gh = jnp.dot(h, whh, preferred_element_type=jnp.float32) + bhh
        r = jax.nn.sigmoid(gi[:, :HIDDEN] + gh[:, :HIDDEN])
        z = jax.nn.sigmoid(gi[:, HIDDEN:2 * HIDDEN] + gh[:, HIDDEN:2 * HIDDEN])
        n = jnp.tanh(gi[:, 2 * HIDDEN:] + r * gh[:, 2 * HIDDEN:])
        h = (1.0 - z) * n + z * h
    mu = jnp.mean(h, axis=-1, keepdims=True)
    var = jnp.mean((h - mu) * (h - mu), axis=-1, keepdims=True)
    ln = (h - mu) * lax.rsqrt(var + 1e-5) * lng_ref[...] + lnb_ref[...]
    out_ref[...] = jnp.dot(ln, fcw_ref[...],
                           preferred_element_type=jnp.float32) + fcb_ref[...]


_gru_call = pl.pallas_call(
    _gru_body,
    grid=(N_NODES // ROWS_PER_BLOCK,),
    in_specs=[
        pl.BlockSpec((ROWS_PER_BLOCK, DEG, HIDDEN), lambda i: (i, 0, 0)),
        pl.BlockSpec((HIDDEN, 3 * HIDDEN), lambda i: (0, 0)),
        pl.BlockSpec((HIDDEN, 3 * HIDDEN), lambda i: (0, 0)),
        pl.BlockSpec((1, 3 * HIDDEN), lambda i: (0, 0)),
        pl.BlockSpec((1, 3 * HIDDEN), lambda i: (0, 0)),
        pl.BlockSpec((1, HIDDEN), lambda i: (0, 0)),
        pl.BlockSpec((1, HIDDEN), lambda i: (0, 0)),
        pl.BlockSpec((HIDDEN, N_CLASSES), lambda i: (0, 0)),
        pl.BlockSpec((1, N_CLASSES), lambda i: (0, 0)),
    ],
    out_specs=pl.BlockSpec((ROWS_PER_BLOCK, N_CLASSES), lambda i: (i, 0)),
    out_shape=jax.ShapeDtypeStruct((N_NODES, N_CLASSES), jnp.float32),
    compiler_params=pltpu.CompilerParams(
        dimension_semantics=("arbitrary",),
    ),
)


def kernel(token_ids, neighbor_idx, emb, W_ih, W_hh, b_ih, b_hh, ln_g, ln_b,
           fc_W, fc_b):
    nbr_flat = neighbor_idx.reshape(-1).astype(jnp.int32)
    nbr_flat = jnp.concatenate(
        [nbr_flat, jnp.zeros((IDX_PAD_CHUNKS * CHUNK - MSGS,), jnp.int32)])
    nbr2 = nbr_flat.reshape(IDX_PAD_CHUNKS, CHUNK)
    msg_flat = _get_sc_gather()(token_ids.astype(jnp.int32), nbr2, emb)
    msg = msg_flat.reshape(ROWS_PAD, DEG, HIDDEN)
    out = _gru_call(
        msg,
        W_ih.T, W_hh.T,
        b_ih.reshape(1, -1), b_hh.reshape(1, -1),
        ln_g.reshape(1, -1), ln_b.reshape(1, -1),
        fc_W.T, fc_b.reshape(1, -1),
    )
    return out
